# baseline (speedup 1.0000x reference)
import jax
import jax.numpy as jnp
from jax import lax
from jax.experimental import pallas as pl
from jax.experimental.pallas import tpu as pltpu
from jax.experimental.pallas import tpu_sc as plsc

HIDDEN = 64
PADW = 128
SEQ = 200
BSZ = 1024

NC = 2
NS = 16
L = 16
NW = NC * NS

NP = 4
NQ = 8
PPW = SEQ // NP
BPW = BSZ // NQ
NB = 3


def _emb_kernel(tok_hbm, ids_hbm, pos_hbm, out_hbm,
                ids_v, pos_v, gb0, gb1, gb2, ob0, ob1, ob2,
                gsem0, gsem1, gsem2, osem0, osem1, osem2):
    wid = lax.axis_index("s") * NC + lax.axis_index("c")
    p = wid // NQ
    q = lax.rem(wid, NQ)
    gbs = (gb0, gb1, gb2)
    obs = (ob0, ob1, ob2)
    gsems = (gsem0, gsem1, gsem2)
    osems = (osem0, osem1, osem2)

    pltpu.sync_copy(ids_hbm.at[p, q], ids_v)
    pltpu.sync_copy(pos_hbm.at[pl.ds(0, SEQ)], pos_v)

    lane = lax.iota(jnp.int32, L)
    rowidx = [c * L + lane for c in range(4)]

    def gather_start(sp, j):
        pltpu.async_copy(tok_hbm.at[ids_v.at[sp]], gbs[j], gsems[j])

    def gather_wait(j):
        pltpu.make_async_copy(tok_hbm.at[pl.ds(0, BPW)], gbs[j],
                              gsems[j]).wait()

    def scatter_start(sp, j):
        pltpu.async_copy(obs[j],
                         out_hbm.at[p * PPW + sp, :, pl.ds(q * BPW, BPW)],
                         osems[j])

    def scatter_wait(sp, j):
        pltpu.make_async_copy(obs[j],
                              out_hbm.at[p * PPW + sp, :,
                                         pl.ds(q * BPW, BPW)],
                              osems[j]).wait()

    def transpose_add(sp, j):
        gb, ob = gbs[j], obs[j]
        s = p * PPW + sp
        pv = [pos_v[s, pl.ds(c * L, L)] for c in range(4)]

        @pl.loop(0, BPW, step=4)
        def _(i0):
            colbase = jnp.full((L,), 0, jnp.int32) + i0
            for di in range(4):
                i = i0 + di
                col = colbase + di
                for c in range(4):
                    val = gb[i, pl.ds(c * L, L)] + pv[c]
                    plsc.store_scatter(ob, [rowidx[c], col], val)

    for j in range(NB):
        gather_start(j, j)
    for sp in range(NB):
        j = sp % NB
        gather_wait(j)
        transpose_add(sp, j)
        scatter_start(sp, j)
        gather_start(sp + NB, j)

    @pl.loop(NB, PPW - NB - 2, step=NB)
    def _(sp0):
        for jj in range(NB):
            sp = sp0 + jj
            gather_wait(jj)
            scatter_wait(sp - NB, jj)
            transpose_add(sp, jj)
            scatter_start(sp, jj)
            gather_start(sp + NB, jj)

    for sp in range(PPW - NB - 2, PPW):
        j = sp % NB
        gather_wait(j)
        scatter_wait(sp - NB, j)
        transpose_add(sp, j)
        scatter_start(sp, j)
        if sp + NB < PPW:
            gather_start(sp + NB, j)
    for sp in range(PPW - NB, PPW):
        scatter_wait(sp, sp % NB)


@jax.jit
def _emb(tok_padded, ids_blk, pos_table):
    mesh = plsc.VectorSubcoreMesh(core_axis_name="c", subcore_axis_name="s")
    f = pl.kernel(
        _emb_kernel,
        out_type=jax.ShapeDtypeStruct((SEQ, HIDDEN, BSZ), jnp.float32),
        mesh=mesh,
        compiler_params=pltpu.CompilerParams(
            needs_layout_passes=False,
            disable_bounds_checks=True,
            disable_semaphore_checks=True,
        ),
        scratch_types=[
            pltpu.VMEM((PPW, BPW), jnp.int32),
            pltpu.VMEM((SEQ, HIDDEN), jnp.float32),
            pltpu.VMEM((BPW, PADW), jnp.float32),
            pltpu.VMEM((BPW, PADW), jnp.float32),
            pltpu.VMEM((BPW, PADW), jnp.float32),
            pltpu.VMEM((HIDDEN, BPW), jnp.float32),
            pltpu.VMEM((HIDDEN, BPW), jnp.float32),
            pltpu.VMEM((HIDDEN, BPW), jnp.float32),
            pltpu.SemaphoreType.DMA,
            pltpu.SemaphoreType.DMA,
            pltpu.SemaphoreType.DMA,
            pltpu.SemaphoreType.DMA,
            pltpu.SemaphoreType.DMA,
            pltpu.SemaphoreType.DMA,
        ],
    )
    return f(tok_padded, ids_blk, pos_table)


def kernel(input_ids, tok_table, pos_table):
    tok_padded = jnp.pad(tok_table, ((0, 0), (0, PADW - HIDDEN)))
    ids_blk = (input_ids.astype(jnp.int32).T
               .reshape(NP, PPW, NQ, BPW).transpose(0, 2, 1, 3))
    out_t = _emb(tok_padded, ids_blk, pos_table)
    return out_t.transpose(2, 0, 1)


# speedup vs baseline: 1.2321x; 1.2321x over previous
import jax
import jax.numpy as jnp
from jax import lax
from jax.experimental import pallas as pl
from jax.experimental.pallas import tpu as pltpu
from jax.experimental.pallas import tpu_sc as plsc

VOCAB = 1000000
HIDDEN = 64
PADW = 128
SEQ = 200
BSZ = 1024

NC = 2
NS = 16
L = 16
NW = NC * NS

SPW = BSZ // NW
G0, G1 = 128, SEQ - 128
NB = 3

TCOLS = 1920
TSTEPS = -(-VOCAB // TCOLS)


def _relayout_kernel(tt_ref, out_ref):
    x = tt_ref[...]
    xt = x.T
    out_ref[...] = jnp.concatenate([xt, xt], axis=1)


@jax.jit
def _relayout(tok_t):
    return pl.pallas_call(
        _relayout_kernel,
        grid=(TSTEPS,),
        in_specs=[pl.BlockSpec((HIDDEN, TCOLS), lambda i: (0, i))],
        out_specs=pl.BlockSpec((TCOLS, PADW), lambda i: (i, 0)),
        out_shape=jax.ShapeDtypeStruct((VOCAB, PADW), jnp.float32),
    )(tok_t)


def _emb_kernel(tok_hbm, ids_hbm, pos_hbm, out_hbm,
                ids_v, pos_v, b0, b1, b2,
                gsem0, gsem1, gsem2, osem0, osem1, osem2):
    wid = lax.axis_index("s") * NC + lax.axis_index("c")
    bufs = (b0, b1, b2)
    gsems = (gsem0, gsem1, gsem2)
    osems = (osem0, osem1, osem2)

    pltpu.sync_copy(ids_hbm.at[pl.ds(wid * SPW, SPW)], ids_v)
    pltpu.sync_copy(pos_hbm.at[pl.ds(0, SEQ)], pos_v)

    def gather_start(s, j):
        pltpu.async_copy(tok_hbm.at[ids_v.at[s, pl.ds(0, G0)]],
                         bufs[j].at[pl.ds(0, G0)], gsems[j])
        pltpu.async_copy(tok_hbm.at[ids_v.at[s, pl.ds(G0, G1)]],
                         bufs[j].at[pl.ds(G0, G1)], gsems[j])

    def gather_wait(j):
        pltpu.make_async_copy(tok_hbm.at[pl.ds(0, SEQ)], bufs[j],
                              gsems[j]).wait()

    def scatter_start(s, j):
        pltpu.async_copy(bufs[j], out_hbm.at[wid * SPW + s], osems[j])

    def scatter_wait(s, j):
        pltpu.make_async_copy(bufs[j], out_hbm.at[wid * SPW + s],
                              osems[j]).wait()

    def add_pos(j):
        buf = bufs[j]

        @pl.loop(0, SEQ, step=2)
        def _(r):
            for rr in range(2):
                row = r + rr
                for c in range(4):
                    sl = pl.ds(c * L, L)
                    buf[row, sl] = buf[row, sl] + pos_v[row, sl]

    for j in range(NB):
        gather_start(j, j)
    for s in range(NB):
        j = s % NB
        gather_wait(j)
        add_pos(j)
        scatter_start(s, j)
        gather_start(s + NB, j)

    @pl.loop(NB, SPW - NB - 2, step=NB)
    def _(s0):
        for jj in range(NB):
            s = s0 + jj
            gather_wait(jj)
            scatter_wait(s - NB, jj)
            add_pos(jj)
            scatter_start(s, jj)
            gather_start(s + NB, jj)

    for s in range(SPW - NB - 2, SPW):
        j = s % NB
        gather_wait(j)
        scatter_wait(s - NB, j)
        add_pos(j)
        scatter_start(s, j)
        if s + NB < SPW:
            gather_start(s + NB, j)
    for s in range(SPW - NB, SPW):
        scatter_wait(s, s % NB)


@jax.jit
def _emb(tok_padded, ids, pos_table):
    mesh = plsc.VectorSubcoreMesh(core_axis_name="c", subcore_axis_name="s")
    f = pl.kernel(
        _emb_kernel,
        out_type=jax.ShapeDtypeStruct((BSZ, SEQ, PADW), jnp.float32),
        mesh=mesh,
        compiler_params=pltpu.CompilerParams(
            disable_bounds_checks=True,
            disable_semaphore_checks=True,
        ),
        scratch_types=[
            pltpu.VMEM((SPW, SEQ), jnp.int32),
            pltpu.VMEM((SEQ, HIDDEN), jnp.float32),
            pltpu.VMEM((SEQ, PADW), jnp.float32),
            pltpu.VMEM((SEQ, PADW), jnp.float32),
            pltpu.VMEM((SEQ, PADW), jnp.float32),
            pltpu.SemaphoreType.DMA,
            pltpu.SemaphoreType.DMA,
            pltpu.SemaphoreType.DMA,
            pltpu.SemaphoreType.DMA,
            pltpu.SemaphoreType.DMA,
            pltpu.SemaphoreType.DMA,
        ],
    )
    return f(tok_padded, ids, pos_table)


def kernel(input_ids, tok_table, pos_table):
    tok_padded = _relayout(tok_table.T)
    out_p = _emb(tok_padded, input_ids.astype(jnp.int32), pos_table)
    return out_p[:, :, :HIDDEN]
